# baseline (speedup 1.0000x reference)
import functools

import jax
import jax.numpy as jnp
from jax import lax
from jax.experimental import pallas as pl
from jax.experimental.pallas import tpu as pltpu
from jax.experimental.pallas import tpu_sc as plsc

_N_CLASSES = 1000
_LANES = 16
_ROWS_PER_BLOCK = 64


@functools.cache
def _make_onehot(n_rows, n_classes, rows_per_block):
    info = plsc.get_sparse_core_info()
    n_workers = info.num_cores * info.num_subcores
    rows_per_w = n_rows // n_workers
    n_blocks = rows_per_w // rows_per_block
    blk_elems = rows_per_block * n_classes
    mesh = plsc.VectorSubcoreMesh(core_axis_name="c", subcore_axis_name="s")

    @functools.partial(
        pl.kernel,
        out_type=jax.ShapeDtypeStruct((n_rows * n_classes,), jnp.float32),
        mesh=mesh,
        scratch_types=[
            pltpu.VMEM((rows_per_w,), jnp.int32),
            pltpu.VMEM((blk_elems,), jnp.float32),
        ],
        compiler_params=pltpu.CompilerParams(needs_layout_passes=False),
    )
    def k(x_hbm, out_hbm, x_v, buf):
        wid = lax.axis_index("s") * info.num_cores + lax.axis_index("c")
        row0 = wid * rows_per_w
        pltpu.sync_copy(x_hbm.at[pl.ds(row0, rows_per_w)], x_v)

        zeros16 = jnp.zeros((_LANES,), jnp.float32)
        ones16 = jnp.ones((_LANES,), jnp.float32)
        iota16 = lax.iota(jnp.int32, _LANES)

        def zero_body(i, carry):
            buf[pl.ds(i * _LANES, _LANES)] = zeros16
            return carry

        lax.fori_loop(0, blk_elems // _LANES, zero_body, 0)

        def offs_for(g, i):
            cols = x_v[pl.ds(g * rows_per_block + i * _LANES, _LANES)]
            rows = iota16 + (i * _LANES)
            return rows * n_classes + cols

        def block_body(g, carry):
            for i in range(rows_per_block // _LANES):
                plsc.store_scatter(buf, [offs_for(g, i)], ones16)
            pltpu.sync_copy(
                buf,
                out_hbm.at[pl.ds((row0 + g * rows_per_block) * n_classes,
                                 blk_elems)],
            )
            for i in range(rows_per_block // _LANES):
                plsc.store_scatter(buf, [offs_for(g, i)], zeros16)
            return carry

        lax.fori_loop(0, n_blocks, block_body, 0)

    return k


def kernel(x):
    b, l = x.shape
    n_rows = b * l
    xf = x.reshape(n_rows).astype(jnp.int32)
    out = _make_onehot(n_rows, _N_CLASSES, _ROWS_PER_BLOCK)(xf)
    return out.reshape(b, l, _N_CLASSES)
